# baseline (speedup 1.0000x reference)
import functools

import jax
import jax.numpy as jnp
from jax import lax
from jax.experimental import pallas as pl
from jax.experimental.pallas import tpu as pltpu
from jax.experimental.pallas import tpu_sc as plsc

_NC = 2
_NS = 16
_NW = _NC * _NS


@functools.partial(jax.jit, static_argnums=(2, 3, 4, 5))
def _sc_embed(mask_flat, W, B, S, A, D):
    N = B * S
    TPW = N // _NW
    QS = S // (_NW // B)
    NQ = _NW // B
    mesh = plsc.VectorSubcoreMesh(core_axis_name="c", subcore_axis_name="s")

    @functools.partial(
        pl.kernel,
        mesh=mesh,
        compiler_params=pltpu.CompilerParams(
            needs_layout_passes=False, use_tc_tiling_on_sc=False
        ),
        out_type=jax.ShapeDtypeStruct((B, A, S, D), jnp.float32),
        scratch_types=[
            pltpu.VMEM((A * D,), jnp.float32),
            pltpu.VMEM((TPW * A,), jnp.float32),
            pltpu.VMEM((1, 1, QS, D), jnp.float32),
            pltpu.VMEM((1, 1, QS, D), jnp.float32),
            pltpu.VMEM((1, 1, QS, D), jnp.float32),
            pltpu.SemaphoreType.DMA,
            pltpu.SemaphoreType.DMA,
            pltpu.SemaphoreType.DMA,
        ],
    )
    def k(m_hbm, w_hbm, out_hbm, w_v, m_v, o_v0, o_v1, o_v2, sem0, sem1, sem2):
        wid = lax.axis_index("s") * _NC + lax.axis_index("c")
        b_idx = wid // NQ
        q_idx = wid % NQ
        s0 = q_idx * QS
        bufs = [o_v0, o_v1, o_v2]
        sems = [sem0, sem1, sem2]
        pltpu.sync_copy(w_hbm, w_v)
        pltpu.sync_copy(m_hbm.at[pl.ds(wid * TPW * A, TPW * A)], m_v)

        for a in range(A):
            u = a % 3
            o_v = bufs[u]
            if a >= 3:
                pltpu.make_async_copy(
                    o_v, out_hbm.at[pl.ds(b_idx, 1), pl.ds(a, 1), pl.ds(s0, QS)], sems[u]
                ).wait()
            wvecs = [w_v[pl.ds(a * D + 16 * j, 16)] for j in range(D // 16)]

            def tbody(i, c, a=a, o_v=o_v, wvecs=wvecs):
                bcs = [
                    plsc.load_gather(
                        m_v, [jnp.full((16,), (8 * i + u) * A + a, jnp.int32)]
                    )
                    for u in range(8)
                ]
                for u in range(8):
                    for j in range(D // 16):
                        o_v[0, 0, 8 * i + u, pl.ds(16 * j, 16)] = (
                            wvecs[j] * bcs[u]
                        )
                return c

            lax.fori_loop(0, QS // 8, tbody, 0)
            pltpu.async_copy(
                o_v, out_hbm.at[pl.ds(b_idx, 1), pl.ds(a, 1), pl.ds(s0, QS)], sems[u]
            )
        for u in range(3):
            pltpu.make_async_copy(
                bufs[u], out_hbm.at[pl.ds(b_idx, 1), pl.ds(0, 1), pl.ds(s0, QS)], sems[u]
            ).wait()

    return k(mask_flat, W)


def kernel(atom_mask, W):
    B, S, A = atom_mask.shape
    D = W.shape[1]
    out = _sc_embed(atom_mask.reshape(B * S * A), W.reshape(A * D), B, S, A, D)
    return out.transpose(0, 2, 1, 3)


# speedup vs baseline: 1.0230x; 1.0230x over previous
import functools

import jax
import jax.numpy as jnp
from jax import lax
from jax.experimental import pallas as pl
from jax.experimental.pallas import tpu as pltpu
from jax.experimental.pallas import tpu_sc as plsc

_NC = 2
_NS = 16
_NW = _NC * _NS


@functools.partial(jax.jit, static_argnums=(2, 3, 4, 5))
def _sc_embed(mask_flat, W, B, S, A, D):
    N = B * S
    TPW = N // _NW
    QS = S // (_NW // B)
    NQ = _NW // B
    mesh = plsc.VectorSubcoreMesh(core_axis_name="c", subcore_axis_name="s")

    @functools.partial(
        pl.kernel,
        mesh=mesh,
        compiler_params=pltpu.CompilerParams(
            needs_layout_passes=False, use_tc_tiling_on_sc=False
        ),
        out_type=jax.ShapeDtypeStruct((B, A, S, D), jnp.float32),
        scratch_types=[
            pltpu.VMEM((A * D,), jnp.float32),
            pltpu.VMEM((TPW * A,), jnp.float32),
            pltpu.VMEM((1, 1, QS, D), jnp.float32),
            pltpu.VMEM((1, 1, QS, D), jnp.float32),
            pltpu.VMEM((1, 1, QS, D), jnp.float32),
            pltpu.SemaphoreType.DMA,
            pltpu.SemaphoreType.DMA,
            pltpu.SemaphoreType.DMA,
        ],
    )
    def k(m_hbm, w_hbm, out_hbm, w_v, m_v, o_v0, o_v1, o_v2, sem0, sem1, sem2):
        wid = lax.axis_index("s") * _NC + lax.axis_index("c")
        b_idx = wid // NQ
        q_idx = wid % NQ
        s0 = q_idx * QS
        bufs = [o_v0, o_v1, o_v2]
        sems = [sem0, sem1, sem2]
        pltpu.sync_copy(w_hbm, w_v)
        pltpu.sync_copy(m_hbm.at[pl.ds(wid * TPW * A, TPW * A)], m_v)

        for a in range(A):
            u = a % 3
            o_v = bufs[u]
            if a >= 3:
                pltpu.make_async_copy(
                    o_v, out_hbm.at[pl.ds(b_idx, 1), pl.ds(a, 1), pl.ds(s0, QS)], sems[u]
                ).wait()
            wvecs = [w_v[pl.ds(a * D + 16 * j, 16)] for j in range(D // 16)]

            def tbody(i, c, a=a, o_v=o_v, wvecs=wvecs):
                bcs = [
                    plsc.load_gather(
                        m_v, [jnp.full((16,), (4 * i + u) * A + a, jnp.int32)]
                    )
                    for u in range(4)
                ]
                for u in range(4):
                    for j in range(D // 16):
                        o_v[0, 0, 4 * i + u, pl.ds(16 * j, 16)] = (
                            wvecs[j] * bcs[u]
                        )
                return c

            lax.fori_loop(0, QS // 4, tbody, 0)
            pltpu.async_copy(
                o_v, out_hbm.at[pl.ds(b_idx, 1), pl.ds(a, 1), pl.ds(s0, QS)], sems[u]
            )
        for u in range(3):
            pltpu.make_async_copy(
                bufs[u], out_hbm.at[pl.ds(b_idx, 1), pl.ds(0, 1), pl.ds(s0, QS)], sems[u]
            ).wait()

    return k(mask_flat, W)


def kernel(atom_mask, W):
    B, S, A = atom_mask.shape
    D = W.shape[1]
    out = _sc_embed(atom_mask.reshape(B * S * A), W.reshape(A * D), B, S, A, D)
    return out.transpose(0, 2, 1, 3)


# speedup vs baseline: 1.0799x; 1.0556x over previous
import functools

import jax
import jax.numpy as jnp
from jax import lax
from jax.experimental import pallas as pl
from jax.experimental.pallas import tpu as pltpu
from jax.experimental.pallas import tpu_sc as plsc

_NC = 2
_NS = 16
_NW = _NC * _NS


@functools.partial(jax.jit, static_argnums=(2, 3, 4, 5))
def _sc_embed(mask_flat, W, B, S, A, D):
    N = B * S
    TPW = N // _NW
    QS = S // (_NW // B)
    NQ = _NW // B
    mesh = plsc.VectorSubcoreMesh(core_axis_name="c", subcore_axis_name="s")

    @functools.partial(
        pl.kernel,
        mesh=mesh,
        compiler_params=pltpu.CompilerParams(
            needs_layout_passes=False, use_tc_tiling_on_sc=False
        ),
        out_type=jax.ShapeDtypeStruct((B, A, S, D), jnp.float32),
        scratch_types=[
            pltpu.VMEM((A * D,), jnp.float32),
            pltpu.VMEM((TPW * A,), jnp.float32),
            pltpu.VMEM((1, 1, QS, D), jnp.float32),
            pltpu.VMEM((1, 1, QS, D), jnp.float32),
            pltpu.VMEM((1, 1, QS, D), jnp.float32),
            pltpu.SemaphoreType.DMA,
            pltpu.SemaphoreType.DMA,
            pltpu.SemaphoreType.DMA,
        ],
    )
    def k(m_hbm, w_hbm, out_hbm, w_v, m_v, o_v0, o_v1, o_v2, sem0, sem1, sem2):
        wid = lax.axis_index("s") * _NC + lax.axis_index("c")
        b_idx = wid // NQ
        q_idx = wid % NQ
        s0 = q_idx * QS
        bufs = [o_v0, o_v1, o_v2]
        sems = [sem0, sem1, sem2]
        pltpu.sync_copy(w_hbm, w_v)
        pltpu.sync_copy(m_hbm.at[pl.ds(wid * TPW * A, TPW * A)], m_v)

        for a in range(A):
            u = a % 3
            o_v = bufs[u]
            if a >= 3:
                pltpu.make_async_copy(
                    o_v, out_hbm.at[pl.ds(b_idx, 1), pl.ds(a, 1), pl.ds(s0, QS)], sems[u]
                ).wait()
            wvecs = [w_v[pl.ds(a * D + 16 * j, 16)] for j in range(D // 16)]

            @plsc.parallel_loop(0, QS // 4, unroll=2)
            def tbody(i, a=a, o_v=o_v, wvecs=wvecs):
                bcs = [
                    plsc.load_gather(
                        m_v, [jnp.full((16,), (4 * i + u) * A + a, jnp.int32)]
                    )
                    for u in range(4)
                ]
                for u in range(4):
                    for j in range(D // 16):
                        o_v[0, 0, 4 * i + u, pl.ds(16 * j, 16)] = (
                            wvecs[j] * bcs[u]
                        )
            pltpu.async_copy(
                o_v, out_hbm.at[pl.ds(b_idx, 1), pl.ds(a, 1), pl.ds(s0, QS)], sems[u]
            )
        for u in range(3):
            pltpu.make_async_copy(
                bufs[u], out_hbm.at[pl.ds(b_idx, 1), pl.ds(0, 1), pl.ds(s0, QS)], sems[u]
            ).wait()

    return k(mask_flat, W)


def kernel(atom_mask, W):
    B, S, A = atom_mask.shape
    D = W.shape[1]
    out = _sc_embed(atom_mask.reshape(B * S * A), W.reshape(A * D), B, S, A, D)
    return out.transpose(0, 2, 1, 3)
